# baseline (speedup 1.0000x reference)
import functools

import jax
import jax.numpy as jnp
from jax import lax
from jax.experimental import pallas as pl
from jax.experimental.pallas import tpu as pltpu
from jax.experimental.pallas import tpu_sc as plsc

NF = 384
G = 16


def kernel(x, indices):
    B, C, H, W = x.shape
    NP = B * NF
    zplane = jnp.zeros((H, W), x.dtype)

    mesh = plsc.VectorSubcoreMesh(core_axis_name="c", subcore_axis_name="s")
    NW = 32
    RPW = NP // NW

    @functools.partial(
        pl.kernel,
        out_type=jax.ShapeDtypeStruct((B, NF, H, W), x.dtype),
        mesh=mesh,
        scratch_types=[
            pltpu.VMEM((RPW,), jnp.int32),
            pltpu.VMEM((G, H, W), x.dtype),
            pltpu.VMEM((H, W), x.dtype),
            pltpu.SemaphoreType.DMA,
            pltpu.SemaphoreType.DMA,
        ],
        compiler_params=pltpu.CompilerParams(
            use_tc_tiling_on_sc=True, needs_layout_passes=False
        ),
    )
    def sc_gather(x_hbm, idx_hbm, z_hbm, out_hbm, idx_v, buf_v, zero_v, gsem, ssem):
        wid = lax.axis_index("c") * 16 + lax.axis_index("s")
        base = wid * RPW
        b = base // NF
        i0 = base % NF
        bC = b * C

        pltpu.sync_copy(idx_hbm.at[pl.ds(i0, RPW)], idx_v)
        pltpu.sync_copy(z_hbm, zero_v)

        def group(g, _):
            k0 = g * G
            iv = idx_v[pl.ds(k0, G)]
            nv = plsc.all_reduce_population_count(iv < C)[0]

            for j in range(G):
                v = iv[j]

                @pl.when(v < C)
                def _fire(j=j, v=v):
                    pltpu.async_copy(x_hbm.at[b, v], buf_v.at[j], gsem)

            def drain(i, _):
                pltpu.make_async_copy(x_hbm.at[0], buf_v.at[0], gsem).wait()
                return 0

            lax.fori_loop(0, nv, drain, 0)

            for j in range(G):
                v = iv[j]

                @pl.when(v < C)
                def _store(j=j):
                    pltpu.async_copy(buf_v.at[j], out_hbm.at[b, i0 + k0 + j], ssem)

                @pl.when(v >= C)
                def _zero(j=j):
                    pltpu.async_copy(zero_v, out_hbm.at[b, i0 + k0 + j], ssem)

            def draw(i, _):
                pltpu.make_async_copy(zero_v, out_hbm.at[b, i0], ssem).wait()
                return 0

            lax.fori_loop(0, G, draw, 0)
            return 0

        lax.fori_loop(0, RPW // G, group, 0)

    return sc_gather(x, indices, zplane)


# speedup vs baseline: 1.0665x; 1.0665x over previous
import functools

import jax
import jax.numpy as jnp
from jax import lax
from jax.experimental import pallas as pl
from jax.experimental.pallas import tpu as pltpu
from jax.experimental.pallas import tpu_sc as plsc

NF = 384
G = 8
NG = 192 // G


def kernel(x, indices):
    B, C, H, W = x.shape
    zplane = jnp.zeros((H, W), x.dtype)

    mesh = plsc.VectorSubcoreMesh(core_axis_name="c", subcore_axis_name="s")

    @functools.partial(
        pl.kernel,
        out_type=jax.ShapeDtypeStruct((B, NF, H, W), x.dtype),
        mesh=mesh,
        scratch_types=[
            pltpu.VMEM((208,), jnp.int32),
            pltpu.VMEM((2, G, H, W), x.dtype),
            pltpu.VMEM((H, W), x.dtype),
            pltpu.SemaphoreType.DMA,
            pltpu.SemaphoreType.DMA,
            pltpu.SemaphoreType.DMA,
            pltpu.SemaphoreType.DMA,
        ],
        compiler_params=pltpu.CompilerParams(
            use_tc_tiling_on_sc=True, needs_layout_passes=False
        ),
    )
    def sc_gather(x_hbm, idx_hbm, z_hbm, out_hbm, idx_v, buf_v, zero_v,
                  gsem0, gsem1, ssem0, ssem1):
        gsems = (gsem0, gsem1)
        ssems = (ssem0, ssem1)
        wid = lax.axis_index("c") * 16 + lax.axis_index("s")
        b = wid // 2
        i0 = (wid % 2) * 192

        pltpu.sync_copy(idx_hbm.at[pl.ds(i0, 192)], idx_v.at[pl.ds(0, 192)])
        pltpu.sync_copy(z_hbm, zero_v)

        def drain_gathers(sem, n):
            def _w(i, _):
                pltpu.make_async_copy(x_hbm.at[0, 0], buf_v.at[0, 0], sem).wait()
                return 0
            lax.fori_loop(0, n, _w, 0)

        def drain_stores(sem, n):
            for _ in range(n):
                pltpu.make_async_copy(zero_v, out_hbm.at[b, i0], sem).wait()

        def fire_gathers(g, bank):
            iv = idx_v[pl.ds(g * G, 16)]
            nv = 0
            for j in range(G):
                v = iv[j]

                @pl.when(v < C)
                def _fire(j=j, v=v, bank=bank):
                    pltpu.async_copy(x_hbm.at[b, v], buf_v.at[bank, j],
                                     gsems[bank])

                nv = nv + jnp.where(v < C, 1, 0)
            return iv, nv

        def fire_stores(g, bank, iv):
            for j in range(G):
                v = iv[j]
                ch = i0 + g * G + j

                @pl.when(v < C)
                def _store(j=j, bank=bank, ch=ch):
                    pltpu.async_copy(buf_v.at[bank, j], out_hbm.at[b, ch],
                                     ssems[bank])

                @pl.when(v >= C)
                def _zero(bank=bank, ch=ch):
                    pltpu.async_copy(zero_v, out_hbm.at[b, ch], ssems[bank])

        iv_nv = [None] * NG
        iv_nv[0] = fire_gathers(0, 0)
        for g in range(1, NG):
            bank = g & 1
            if g >= 2:
                drain_stores(ssems[bank], G)
            iv_nv[g] = fire_gathers(g, bank)
            piv, pnv = iv_nv[g - 1]
            drain_gathers(gsems[1 - bank], pnv)
            fire_stores(g - 1, 1 - bank, piv)
        liv, lnv = iv_nv[NG - 1]
        lbank = (NG - 1) & 1
        drain_gathers(gsems[lbank], lnv)
        fire_stores(NG - 1, lbank, liv)
        drain_stores(ssems[0], G)
        drain_stores(ssems[1], G)

    return sc_gather(x, indices, zplane)
